# baseline (speedup 1.0000x reference)
import jax
import jax.numpy as jnp
from jax import lax
from jax.experimental import pallas as pl
from jax.experimental.pallas import tpu as pltpu

_F, _V, _D_NAME, _D_VAL = 100, 100, 16, 16
_ROWV = _F * _D_VAL
_BBLK = 512


def _emb_kernel(fv_ref, tab_ref, w_ref, b_ref, idx_ref, val_ref, emb_ref):
    idxs = idx_ref[...]
    vio = lax.broadcasted_iota(jnp.int32, (_V, _F), 0)
    oh_t = (vio == idxs).astype(jnp.float32)
    emb_ref[...] = lax.dot_general(
        oh_t, tab_ref[...], (((0,), (0,)), ((), ())),
        preferred_element_type=jnp.float32)

    jio = lax.broadcasted_iota(jnp.int32, (_F, _ROWV), 1)
    fio = lax.broadcasted_iota(jnp.int32, (_F, _ROWV), 0)
    e_mat = ((jio // _D_VAL) == fio).astype(jnp.float32)
    jio2 = lax.broadcasted_iota(jnp.int32, (_D_VAL, _ROWV), 1)
    cio = lax.broadcasted_iota(jnp.int32, (_D_VAL, _ROWV), 0)
    g_mat = ((jio2 % _D_VAL) == cio).astype(jnp.float32)
    scale_row = lax.dot_general(
        w_ref[...].T, g_mat, (((1,), (0,)), ((), ())),
        preferred_element_type=jnp.float32)
    bias_row = lax.dot_general(
        b_ref[...], g_mat, (((1,), (0,)), ((), ())),
        preferred_element_type=jnp.float32)
    a_mat = e_mat * scale_row

    val_ref[...] = lax.dot_general(
        fv_ref[...], a_mat, (((1,), (0,)), ((), ())),
        preferred_element_type=jnp.float32) + bias_row


def kernel(feature_values, name_table, W, b, name_indices):
    batch = feature_values.shape[0]
    b2 = b.reshape(1, _D_VAL)
    idx2 = name_indices.reshape(1, _F).astype(jnp.int32)
    val2d, name_emb = pl.pallas_call(
        _emb_kernel,
        grid=(batch // _BBLK,),
        in_specs=[
            pl.BlockSpec((_BBLK, _F), lambda i: (i, 0)),
            pl.BlockSpec((_V, _D_NAME), lambda i: (0, 0)),
            pl.BlockSpec((_D_VAL, 1), lambda i: (0, 0)),
            pl.BlockSpec((1, _D_VAL), lambda i: (0, 0)),
            pl.BlockSpec((1, _F), lambda i: (0, 0)),
        ],
        out_specs=[
            pl.BlockSpec((_BBLK, _ROWV), lambda i: (i, 0)),
            pl.BlockSpec((_F, _D_NAME), lambda i: (0, 0)),
        ],
        out_shape=[
            jax.ShapeDtypeStruct((batch, _ROWV), jnp.float32),
            jax.ShapeDtypeStruct((_F, _D_NAME), jnp.float32),
        ],
    )(feature_values, name_table, W, b2, idx2)
    name3d = jnp.broadcast_to(name_emb[None], (batch, _F, _D_NAME))
    return jnp.concatenate([name3d, val2d.reshape(batch, _F, _D_VAL)], axis=-1)
